# baseline (speedup 1.0000x reference)
import functools

import jax
import jax.numpy as jnp
from jax import lax
from jax.experimental import pallas as pl
from jax.experimental.pallas import tpu as pltpu, tpu_sc as plsc

N = 10000
NP = 10240
W = 320
NC = 20
CAP = 80
RB = 40
NBATCH = N // RB
H = 128
HCOLS = W // 2
FCAP = HCOLS * CAP + 2 * 128
EB = 128

_mesh = plsc.VectorSubcoreMesh(core_axis_name="c", subcore_axis_name="s")
_sc_params = pltpu.CompilerParams(needs_layout_passes=False,
                                  use_tc_tiling_on_sc=False)


def _wid():
    return lax.axis_index("s") * 2 + lax.axis_index("c")


@functools.partial(
    pl.kernel, mesh=_mesh,
    out_type=[
        jax.ShapeDtypeStruct((32, W * CAP), jnp.int32),
        jax.ShapeDtypeStruct((32, W * CAP), jnp.float32),
        jax.ShapeDtypeStruct((32, W), jnp.int32),
    ],
    scratch_types=[
        pltpu.VMEM((2, RB, W), jnp.float32),
        pltpu.VMEM((W * CAP,), jnp.int32),
        pltpu.VMEM((W * CAP,), jnp.float32),
        pltpu.VMEM((W,), jnp.int32),
        pltpu.SemaphoreType.DMA,
        pltpu.SemaphoreType.DMA,
    ],
    compiler_params=_sc_params,
)
def _extract(a_hbm, eidx_hbm, evals_hbm, cnts_hbm,
             stage, idxb, valb, cntb, sem0, sem1):
    t = _wid()
    w0 = jnp.where(t == 31, N - W, W * t)
    zero16i = jnp.zeros((16,), jnp.int32)
    zero16f = jnp.zeros((16,), jnp.float32)

    def zi(q, _):
        idxb[pl.ds(16 * q, 16)] = zero16i
        valb[pl.ds(16 * q, 16)] = zero16f
        return 0
    lax.fori_loop(0, W * CAP // 16, zi, 0)

    sems = (sem0, sem1)

    def start(b, buf):
        pltpu.async_copy(a_hbm.at[pl.ds(b * RB, RB), pl.ds(w0, W)],
                         stage.at[buf], sems[buf])

    def wait(buf):
        pltpu.make_async_copy(a_hbm.at[pl.ds(0, RB), pl.ds(w0, W)],
                              stage.at[buf], sems[buf]).wait()

    iota = lax.iota(jnp.int32, 16)
    iota_cap = iota * CAP

    def process(b, buf, offs):
        def row_body(r, offs):
            ivec = jnp.full((16,), b * RB + r, jnp.int32)
            new = []
            for k in range(NC):
                v = stage[buf, r, pl.ds(16 * k, 16)]
                m = v > 0.0
                pos = offs[k] + (16 * k * CAP) + iota_cap
                plsc.store_scatter(idxb, [pos], ivec, mask=m)
                plsc.store_scatter(valb, [pos], v, mask=m)
                new.append(offs[k] + m.astype(jnp.int32))
            return tuple(new)
        return lax.fori_loop(0, RB, row_body, offs)

    start(0, 0)

    def outer(bb, offs):
        b0 = 2 * bb
        wait(0)

        @pl.when(b0 + 1 < NBATCH)
        def _():
            start(b0 + 1, 1)
        offs = process(b0, 0, offs)
        wait(1)

        @pl.when(b0 + 2 < NBATCH)
        def _():
            start(b0 + 2, 0)
        offs = process(b0 + 1, 1, offs)
        return offs

    offs0 = tuple(jnp.zeros((16,), jnp.int32) for _ in range(NC))
    offs = lax.fori_loop(0, NBATCH // 2, outer, offs0)

    for k in range(NC):
        cntb[pl.ds(16 * k, 16)] = offs[k]
    pltpu.sync_copy(idxb, eidx_hbm.at[t])
    pltpu.sync_copy(valb, evals_hbm.at[t])
    pltpu.sync_copy(cntb, cnts_hbm.at[t])


@functools.partial(
    pl.kernel, mesh=_mesh,
    out_type=jax.ShapeDtypeStruct((32, W, H), jnp.float32),
    scratch_types=[
        pltpu.VMEM((HCOLS * CAP,), jnp.int32),
        pltpu.VMEM((HCOLS * CAP,), jnp.float32),
        pltpu.VMEM((W,), jnp.int32),
        pltpu.VMEM((FCAP,), jnp.int32),
        pltpu.VMEM((FCAP,), jnp.float32),
        pltpu.VMEM((FCAP,), jnp.int32),
        pltpu.VMEM((EB, H), jnp.float32),
        pltpu.VMEM((HCOLS, H), jnp.float32),
        pltpu.SemaphoreType.DMA,
    ],
    compiler_params=_sc_params,
)
def _conv(eidx_hbm, evals_hbm, cnts_hbm, xin_hbm, acc_hbm,
          ridx, rval, cntb, fidx, fval, fjl, stage, outb, sem):
    t = _wid()
    iota = lax.iota(jnp.int32, 16)
    zero16i = jnp.zeros((16,), jnp.int32)
    zero16f = jnp.zeros((16,), jnp.float32)

    pltpu.sync_copy(cnts_hbm.at[t], cntb)

    for half in range(2):
        c0 = half * HCOLS
        pltpu.sync_copy(eidx_hbm.at[t, pl.ds(c0 * CAP, HCOLS * CAP)], ridx)
        pltpu.sync_copy(evals_hbm.at[t, pl.ds(c0 * CAP, HCOLS * CAP)], rval)

        def zo(q, _):
            for sss in range(H // 16):
                outb[q, pl.ds(16 * sss, 16)] = zero16f
            return 0
        lax.fori_loop(0, HCOLS, zo, 0)

        def p1_body(k, fcur):
            cv = cntb[pl.ds(c0 + 16 * k, 16)]
            for l in range(16):
                c = 16 * k + l
                cvec = jnp.full((16,), c, jnp.int32)
                for q in range(CAP // 16):
                    src = c * CAP + 16 * q
                    fidx[pl.ds(fcur + 16 * q, 16)] = ridx[pl.ds(src, 16)]
                    fval[pl.ds(fcur + 16 * q, 16)] = rval[pl.ds(src, 16)]
                    fjl[pl.ds(fcur + 16 * q, 16)] = cvec
                fcur = fcur + cv[l]
            return fcur

        fcur = lax.fori_loop(0, HCOLS // 16, p1_body, jnp.int32(0))

        for q in range(EB // 16):
            fidx[pl.ds(fcur + 16 * q, 16)] = zero16i
            fval[pl.ds(fcur + 16 * q, 16)] = zero16f
            fjl[pl.ds(fcur + 16 * q, 16)] = zero16i

        nb = lax.div(fcur + (EB - 1), jnp.int32(EB))

        def batch_body(b, _):
            pltpu.async_copy(
                xin_hbm.at[fidx.at[pl.ds(b * EB, EB)]], stage, sem).wait()

            def grp_body(g, _):
                e0 = b * EB + 16 * g
                av = fval[pl.ds(e0, 16)]
                jv = fjl[pl.ds(e0, 16)]
                for l in range(16):
                    a = av[l]
                    jrow = jnp.full((16,), jv[l], jnp.int32)
                    for s in range(H // 16):
                        row = stage[16 * g + l, pl.ds(16 * s, 16)]
                        plsc.addupdate_scatter(
                            outb, [jrow, 16 * s + iota], row * a)
                return 0

            lax.fori_loop(0, EB // 16, grp_body, 0)
            return 0

        lax.fori_loop(0, nb, batch_body, 0)
        pltpu.sync_copy(outb, acc_hbm.at[t, pl.ds(c0, HCOLS)])


def _full(shape):
    nz = tuple(0 for _ in shape)
    return pl.BlockSpec(shape, lambda *_, _nz=nz: _nz)


def _node_select(flat):
    return jnp.concatenate(
        [flat[:9920], flat[10160:10240],
         jnp.zeros((240,) + flat.shape[1:], flat.dtype)], axis=0)


def _prep_kernel(ev_ref, x_ref, w1_ref, b1_ref, wc_ref, dinv_ref, xin_ref):
    ones = jnp.ones((CAP, 1), jnp.float32)
    deg_col = jax.lax.dot_general(
        ev_ref[...], ones, (((1,), (0,)), ((), ())),
        preferred_element_type=jnp.float32)
    deg = _node_select(deg_col) + 1.0
    dinv = jax.lax.rsqrt(deg)
    dinvb = jnp.broadcast_to(dinv, dinv_ref.shape)
    dinv_ref[...] = dinvb
    feat = jnp.dot(x_ref[...], w1_ref[...],
                   preferred_element_type=jnp.float32) + b1_ref[...]
    xin_ref[...] = dinvb * jnp.dot(feat, wc_ref[...],
                                   preferred_element_type=jnp.float32)


def _post_kernel(acc_ref, xin_ref, dinv_ref, bc_ref, wm_ref, bm_ref,
                 g_ref, be_ref, wn_ref, out_ref, *, compute_next):
    dinv = dinv_ref[...]
    accn = _node_select(acc_ref[...])
    h = dinv * (accn + xin_ref[...]) + bc_ref[...]
    y = jnp.dot(h, wm_ref[...], preferred_element_type=jnp.float32) + bm_ref[...]
    mu = jnp.mean(y, axis=-1, keepdims=True)
    var = jnp.mean((y - mu) ** 2, axis=-1, keepdims=True)
    yn = (y - mu) * jax.lax.rsqrt(var + 1e-5)
    act = jnp.maximum(yn * g_ref[...] + be_ref[...], 0.0)
    if compute_next:
        out_ref[...] = dinv * jnp.dot(act, wn_ref[...],
                                      preferred_element_type=jnp.float32)
    else:
        out_ref[...] = act


def kernel(X, adj_A, lin1_W, lin1_b, conv0_W, conv0_b, mlp0_W, mlp0_b,
              ln0_g, ln0_b, conv1_W, conv1_b, mlp1_W, mlp1_b, ln1_g, ln1_b):
    h = H
    eidx, evals, cnts = _extract(adj_A)
    ev2 = evals.reshape(32 * W, CAP)

    x2p = jnp.pad(X[0], ((0, NP - N), (0, 0)))

    dinvb, xin0 = pl.pallas_call(
        _prep_kernel,
        in_specs=[_full((NP, CAP)), _full((NP, x2p.shape[1])),
                  _full((x2p.shape[1], h)), _full((1, h)), _full((h, h))],
        out_specs=[_full((NP, h)), _full((NP, h))],
        out_shape=[jax.ShapeDtypeStruct((NP, h), jnp.float32),
                   jax.ShapeDtypeStruct((NP, h), jnp.float32)],
    )(ev2, x2p, lin1_W, lin1_b.reshape(1, h), conv0_W)

    def post(acc, xin, bc, wm, bm, lg, lb, wn, compute_next):
        return pl.pallas_call(
            functools.partial(_post_kernel, compute_next=compute_next),
            in_specs=[_full((NP, h)), _full((NP, h)), _full((NP, h)),
                      _full((1, h)), _full((h, h)), _full((1, h)),
                      _full((1, h)), _full((1, h)), _full((h, h))],
            out_specs=_full((NP, h)),
            out_shape=jax.ShapeDtypeStruct((NP, h), jnp.float32),
        )(acc.reshape(NP, h), xin, dinvb, bc.reshape(1, h), wm,
          bm.reshape(1, h), lg.reshape(1, h), lb.reshape(1, h), wn)

    acc0 = _conv(eidx, evals, cnts, xin0)
    xin1 = post(acc0, xin0, conv0_b, mlp0_W, mlp0_b, ln0_g, ln0_b,
                conv1_W, compute_next=True)
    acc1 = _conv(eidx, evals, cnts, xin1)
    z = post(acc1, xin1, conv1_b, mlp1_W, mlp1_b, ln1_g, ln1_b,
             conv1_W, compute_next=False)

    return (z[:N][None], adj_A)


# speedup vs baseline: 5.2714x; 5.2714x over previous
import functools

import jax
import jax.numpy as jnp
from jax.experimental import pallas as pl

BI = 400
BI0 = 200


def _deg_feat_kernel(a_ref, x_ref, w1_ref, b1_ref, deg_ref, feat_ref, abf_ref):
    i = pl.program_id(0)

    @pl.when(i == 0)
    def _():
        deg_ref[...] = jnp.zeros_like(deg_ref)

    a = a_ref[...]
    abf_ref[...] = a.astype(jnp.bfloat16)
    ones = jnp.ones((a.shape[0], 8), jnp.float32)
    deg_ref[...] += jax.lax.dot_general(
        a, ones, (((0,), (0,)), ((), ())), preferred_element_type=jnp.float32)
    feat_ref[...] = (
        jnp.dot(x_ref[...], w1_ref[...], preferred_element_type=jnp.float32)
        + b1_ref[...])


def _prep_kernel(deg_ref, feat_ref, wc_ref, dinv_ref, xin_ref):
    deg = deg_ref[...][:, :1] + 1.0
    dinv = jax.lax.rsqrt(deg)
    dinvb = jnp.broadcast_to(dinv, dinv_ref.shape)
    dinv_ref[...] = dinvb
    xin_ref[...] = dinvb * jnp.dot(
        feat_ref[...], wc_ref[...], preferred_element_type=jnp.float32)


def _conv_kernel(a_ref, xin_ref, dinv_ref, bc_ref, wm_ref, bm_ref, g_ref,
                 be_ref, wn_ref, out_ref, *, nsteps, compute_next):
    i = pl.program_id(0)

    @pl.when(i == 0)
    def _():
        out_ref[...] = jnp.zeros_like(out_ref)

    a = a_ref[...]
    xb = xin_ref[pl.ds(i * BI, BI), :].astype(jnp.bfloat16)
    out_ref[...] += jax.lax.dot_general(
        a, xb, (((0,), (0,)), ((), ())), preferred_element_type=jnp.float32)

    @pl.when(i == nsteps - 1)
    def _():
        dinv = dinv_ref[...]
        h = dinv * (out_ref[...] + xin_ref[...]) + bc_ref[...]
        y = jnp.dot(h, wm_ref[...], preferred_element_type=jnp.float32) + bm_ref[...]
        mu = jnp.mean(y, axis=-1, keepdims=True)
        var = jnp.mean((y - mu) ** 2, axis=-1, keepdims=True)
        yn = (y - mu) * jax.lax.rsqrt(var + 1e-5)
        act = jnp.maximum(yn * g_ref[...] + be_ref[...], 0.0)
        if compute_next:
            out_ref[...] = dinv * jnp.dot(
                act, wn_ref[...], preferred_element_type=jnp.float32)
        else:
            out_ref[...] = act


def _full(shape):
    nz = tuple(0 for _ in shape)
    return pl.BlockSpec(shape, lambda *_, _nz=nz: _nz)


def kernel(X, adj_A, lin1_W, lin1_b, conv0_W, conv0_b, mlp0_W, mlp0_b,
           ln0_g, ln0_b, conv1_W, conv1_b, mlp1_W, mlp1_b, ln1_g, ln1_b):
    n = adj_A.shape[0]
    h = lin1_W.shape[1]
    nsteps = n // BI
    x2 = X[0]
    g = x2.shape[1]

    nsteps0 = n // BI0
    deg8, feat, abf = pl.pallas_call(
        _deg_feat_kernel,
        grid=(nsteps0,),
        in_specs=[
            pl.BlockSpec((BI0, n), lambda i: (i, 0)),
            pl.BlockSpec((BI0, g), lambda i: (i, 0)),
            _full((g, h)),
            _full((1, h)),
        ],
        out_specs=[_full((n, 8)), pl.BlockSpec((BI0, h), lambda i: (i, 0)),
                   pl.BlockSpec((BI0, n), lambda i: (i, 0))],
        out_shape=[
            jax.ShapeDtypeStruct((n, 8), jnp.float32),
            jax.ShapeDtypeStruct((n, h), jnp.float32),
            jax.ShapeDtypeStruct((n, n), jnp.bfloat16),
        ],
    )(adj_A, x2, lin1_W, lin1_b.reshape(1, h))

    dinvb, xin0 = pl.pallas_call(
        _prep_kernel,
        in_specs=[_full((n, 8)), _full((n, h)), _full((h, h))],
        out_specs=[_full((n, h)), _full((n, h))],
        out_shape=[
            jax.ShapeDtypeStruct((n, h), jnp.float32),
            jax.ShapeDtypeStruct((n, h), jnp.float32),
        ],
    )(deg8, feat, conv0_W)

    def conv(a, xin, bc, wm, bm, lg, lb, wn, compute_next):
        return pl.pallas_call(
            functools.partial(_conv_kernel, nsteps=nsteps,
                              compute_next=compute_next),
            grid=(nsteps,),
            in_specs=[
                pl.BlockSpec((BI, n), lambda i: (i, 0)),
                _full((n, h)),
                _full((n, h)),
                _full((1, h)),
                _full((h, h)),
                _full((1, h)),
                _full((1, h)),
                _full((1, h)),
                _full((h, h)),
            ],
            out_specs=_full((n, h)),
            out_shape=jax.ShapeDtypeStruct((n, h), jnp.float32),
        )(a, xin, dinvb, bc.reshape(1, h), wm, bm.reshape(1, h),
          lg.reshape(1, h), lb.reshape(1, h), wn)

    xin1 = conv(abf, xin0, conv0_b, mlp0_W, mlp0_b, ln0_g, ln0_b,
                conv1_W, compute_next=True)
    z = conv(abf, xin1, conv1_b, mlp1_W, mlp1_b, ln1_g, ln1_b,
             conv1_W, compute_next=False)

    return (z[None], adj_A)
